# baseline (speedup 1.0000x reference)
import jax
import jax.numpy as jnp
from jax import lax
from jax.experimental import pallas as pl
from jax.experimental.pallas import tpu as pltpu

B, L, H, R, K = 4, 1024, 512, 7, 10
N = B * L
NUM_LAYERS = 4
PAD = 8
PL_ROWS = L + 2 * PAD
OFFSETS = (-3, -2, -1, 1, 2, 3)
C = 256
CPB = L // C
TR = 128
ACH = L // TR
EPS = 1e-5
BIG = 3.0e38

_DNT = (((1,), (1,)), ((), ()))


def _gear_body(ca_ref, wproj_ref, wl_ref, ws_ref, vecs_ref, out_ref,
               hb_s, at_s, p6_s, d_s):
    l = pl.program_id(0)
    f32 = jnp.float32
    bf16 = jnp.bfloat16

    @pl.when(l == 0)
    def _init():
        iot0 = lax.broadcasted_iota(jnp.int32, (L, TR), 0)
        iot1 = lax.broadcasted_iota(jnp.int32, (L, TR), 1)
        iot8 = lax.broadcasted_iota(jnp.int32, (8, TR), 0)
        GRP = L // 8

        def _stream_argmin(prev_am):
            def g_body(g, carry):
                runm, runi = carry
                rowid = iot8 + g * 8
                v = d_s[pl.ds(g * 8, 8), :]
                v = jnp.where(rowid == prev_am, BIG, v)
                d_s[pl.ds(g * 8, 8), :] = v
                lt = v < runm
                runi = jnp.where(lt, rowid, runi)
                runm = jnp.minimum(v, runm)
                return runm, runi

            runm, runi = lax.fori_loop(
                0, GRP, g_body,
                (jnp.full((8, TR), BIG, f32), jnp.zeros((8, TR), jnp.int32)))
            mm = jnp.min(runm, axis=0, keepdims=True)
            am = jnp.min(jnp.where(runm == mm, runi, L), axis=0,
                         keepdims=True)
            return am

        def _per_batch(b, carry):
            x3 = ca_ref[b]
            xo = jnp.concatenate(
                [x3, jnp.ones((L, 1), f32), jnp.zeros((L, 4), f32)], axis=1)
            h0 = jnp.dot(xo, wproj_ref[...], preferred_element_type=f32)
            hb_s[b, 0:PAD, :] = jnp.zeros((PAD, H), bf16)
            hb_s[b, PAD + L:PL_ROWS, :] = jnp.zeros((PAD, H), bf16)
            hb_s[b, PAD:PAD + L, :] = h0.astype(bf16)

            rows = [jnp.transpose(x3[:, cd:cd + 1]) for cd in range(3)]

            for ci in range(ACH):
                i0 = ci * TR
                d2 = jnp.zeros((L, TR), f32)
                for cd in range(3):
                    col = x3[:, cd:cd + 1]
                    row = rows[cd][:, i0:i0 + TR]
                    df = col - row
                    d2 = d2 + df * df
                self_oh = iot0 == (iot1 + i0)
                d_s[...] = jnp.where(self_oh, BIG, jnp.sqrt(d2))
                am = jnp.full((1, TR), -1, jnp.int32)
                for t in range(K):
                    am = _stream_argmin(am)
                at = ((d_s[...] == BIG) & jnp.logical_not(self_oh)) \
                    | (iot0 == am)
                at_s[b, :, i0:i0 + TR] = at.astype(bf16)
            return carry

        lax.fori_loop(0, B, _per_batch, 0)

    ones_row = jnp.ones((1, C), f32)
    bias = vecs_ref[0, 0:1, :] + vecs_ref[0, 1:2, :]

    s1 = jnp.zeros((1, H), f32)
    s2 = jnp.zeros((1, H), f32)
    for b in range(B):
        p6_s[...] = lax.dot_general(
            hb_s[b, PAD:PAD + L, :], wl_ref[0, :, 6 * H:7 * H], _DNT,
            preferred_element_type=f32).astype(jnp.bfloat16)
        for cj in range(CPB):
            r0 = PAD + cj * C
            acc = lax.dot_general(hb_s[b, r0:r0 + C, :], ws_ref[0], _DNT,
                                  preferred_element_type=f32) + bias
            for r, off in enumerate(OFFSETS):
                acc = acc + lax.dot_general(
                    hb_s[b, r0 - off:r0 - off + C, :],
                    wl_ref[0, :, r * H:(r + 1) * H], _DNT,
                    preferred_element_type=f32)
            acc = acc + jnp.dot(at_s[b, cj * C:(cj + 1) * C, :], p6_s[...],
                                preferred_element_type=f32)
            out_ref[b, cj * C:(cj + 1) * C, :] = acc
            s1 = s1 + jnp.dot(ones_row, acc, preferred_element_type=f32)
            s2 = s2 + jnp.dot(ones_row, acc * acc, preferred_element_type=f32)

    m1 = s1 * (1.0 / N)
    v1 = s2 * (1.0 / N) - m1 * m1
    inv1 = lax.rsqrt(v1 + EPS)
    sc1 = vecs_ref[0, 2:3, :] * inv1
    sh1 = vecs_ref[0, 3:4, :] - m1 * sc1

    t1 = jnp.zeros((1, H), f32)
    t2 = jnp.zeros((1, H), f32)
    for b in range(B):
        for cj in range(CPB):
            r0 = PAD + cj * C
            y = (jnp.maximum(out_ref[b, cj * C:(cj + 1) * C, :] * sc1 + sh1,
                             0.0)
                 + hb_s[b, r0:r0 + C, :].astype(f32))
            out_ref[b, cj * C:(cj + 1) * C, :] = y
            t1 = t1 + jnp.dot(ones_row, y, preferred_element_type=f32)
            t2 = t2 + jnp.dot(ones_row, y * y, preferred_element_type=f32)

    m2 = t1 * (1.0 / N)
    v2 = t2 * (1.0 / N) - m2 * m2
    inv2 = lax.rsqrt(v2 + EPS)
    sc2 = vecs_ref[0, 4:5, :] * inv2
    sh2 = vecs_ref[0, 5:6, :] - m2 * sc2

    for b in range(B):
        for cj in range(CPB):
            r0 = PAD + cj * C
            z = out_ref[b, cj * C:(cj + 1) * C, :] * sc2 + sh2
            out_ref[b, cj * C:(cj + 1) * C, :] = z
            hb_s[b, r0:r0 + C, :] = z.astype(jnp.bfloat16)


def kernel(n_coords, ca_coords, c_coords, params):
    f32 = jnp.float32
    bf16 = jnp.bfloat16
    ca = ca_coords.astype(f32)
    wproj = jnp.concatenate([params["W_proj"].T.astype(f32),
                             params["b_proj"][None, :].astype(f32),
                             jnp.zeros((4, H), f32)], axis=0)
    wl = jnp.stack([params[f"W_lin{i}"].astype(bf16)
                    for i in range(NUM_LAYERS)])
    ws = jnp.stack([params[f"W_self{i}"].astype(bf16)
                    for i in range(NUM_LAYERS)])
    z = jnp.zeros((H,), f32)
    vecs = jnp.stack([
        jnp.stack([params[f"b_lin{i}"], params[f"b_self{i}"],
                   params[f"g_in{i}"], params[f"b_in{i}"],
                   params[f"g_out{i}"], params[f"b_out{i}"], z, z]).astype(f32)
        for i in range(NUM_LAYERS)])

    return pl.pallas_call(
        _gear_body,
        grid=(NUM_LAYERS,),
        in_specs=[
            pl.BlockSpec((B, L, 3), lambda l: (0, 0, 0)),
            pl.BlockSpec((8, H), lambda l: (0, 0)),
            pl.BlockSpec((1, H, R * H), lambda l: (l, 0, 0)),
            pl.BlockSpec((1, H, H), lambda l: (l, 0, 0)),
            pl.BlockSpec((1, 8, H), lambda l: (l, 0, 0)),
        ],
        out_specs=pl.BlockSpec((B, L, H), lambda l: (0, 0, 0)),
        out_shape=jax.ShapeDtypeStruct((B, L, H), f32),
        scratch_shapes=[
            pltpu.VMEM((B, PL_ROWS, H), jnp.bfloat16),
            pltpu.VMEM((B, L, L), jnp.bfloat16),
            pltpu.VMEM((L, H), jnp.bfloat16),
            pltpu.VMEM((L, TR), jnp.float32),
        ],
        compiler_params=pltpu.CompilerParams(
            dimension_semantics=("arbitrary",),
            vmem_limit_bytes=64 * 1024 * 1024,
        ),
    )(ca, wproj, wl, ws, vecs)


# speedup vs baseline: 1.3012x; 1.3012x over previous
import jax
import jax.numpy as jnp
from jax import lax
from jax.experimental import pallas as pl
from jax.experimental.pallas import tpu as pltpu

B, L, H, R, K = 4, 1024, 512, 7, 10
N = B * L
NUM_LAYERS = 4
PAD = 8
PL_ROWS = L + 2 * PAD
OFFSETS = (-3, -2, -1, 1, 2, 3)
C = 256
CPB = L // C
TR = 128
ACH = L // TR
EPS = 1e-5
BIG = 3.0e38

_DNT = (((1,), (1,)), ((), ()))


def _gear_body(ca_ref, wproj_ref, wl_ref, ws_ref, vecs_ref, out_ref,
               hb_s, at_s, p6_s):
    l = pl.program_id(0)
    f32 = jnp.float32
    bf16 = jnp.bfloat16

    @pl.when(l == 0)
    def _init():
        iot0 = lax.broadcasted_iota(jnp.int32, (L, TR), 0)
        iot1 = lax.broadcasted_iota(jnp.int32, (L, TR), 1)

        def _per_batch(b, carry):
            x3 = ca_ref[b]
            xo = jnp.concatenate(
                [x3, jnp.ones((L, 1), f32), jnp.zeros((L, 4), f32)], axis=1)
            h0 = jnp.dot(xo, wproj_ref[...], preferred_element_type=f32)
            hb_s[b, 0:PAD, :] = jnp.zeros((PAD, H), bf16)
            hb_s[b, PAD + L:PL_ROWS, :] = jnp.zeros((PAD, H), bf16)
            hb_s[b, PAD:PAD + L, :] = h0.astype(bf16)

            rows = [jnp.transpose(x3[:, cd:cd + 1]) for cd in range(3)]

            for ci in range(ACH):
                i0 = ci * TR
                d2 = jnp.zeros((L, TR), f32)
                for cd in range(3):
                    col = x3[:, cd:cd + 1]
                    row = rows[cd][:, i0:i0 + TR]
                    df = col - row
                    d2 = d2 + df * df
                self_oh = iot0 == (iot1 + i0)
                d = jnp.where(self_oh, BIG, jnp.sqrt(d2))
                am = jnp.full((1, TR), -1, jnp.int32)
                for t in range(K):
                    d = jnp.where(iot0 == am, BIG, d)
                    m = jnp.min(d, axis=0, keepdims=True)
                    sel = jnp.where(d == m, iot0, L)
                    am = jnp.min(sel, axis=0, keepdims=True)
                at = ((d == BIG) & jnp.logical_not(self_oh)) | (iot0 == am)
                at_s[b, :, i0:i0 + TR] = at.astype(bf16)
            return carry

        lax.fori_loop(0, B, _per_batch, 0)

    ones_row = jnp.ones((1, C), f32)
    bias = vecs_ref[0, 0:1, :] + vecs_ref[0, 1:2, :]

    s1 = jnp.zeros((1, H), f32)
    s2 = jnp.zeros((1, H), f32)
    for b in range(B):
        p6_s[...] = lax.dot_general(
            hb_s[b, PAD:PAD + L, :], wl_ref[0, :, 6 * H:7 * H], _DNT,
            preferred_element_type=f32).astype(jnp.bfloat16)
        for cj in range(CPB):
            r0 = PAD + cj * C
            acc = lax.dot_general(hb_s[b, r0:r0 + C, :], ws_ref[0], _DNT,
                                  preferred_element_type=f32) + bias
            for r, off in enumerate(OFFSETS):
                acc = acc + lax.dot_general(
                    hb_s[b, r0 - off:r0 - off + C, :],
                    wl_ref[0, :, r * H:(r + 1) * H], _DNT,
                    preferred_element_type=f32)
            acc = acc + jnp.dot(at_s[b, cj * C:(cj + 1) * C, :], p6_s[...],
                                preferred_element_type=f32)
            out_ref[b, cj * C:(cj + 1) * C, :] = acc
            s1 = s1 + jnp.dot(ones_row, acc, preferred_element_type=f32)
            s2 = s2 + jnp.dot(ones_row, acc * acc, preferred_element_type=f32)

    m1 = s1 * (1.0 / N)
    v1 = s2 * (1.0 / N) - m1 * m1
    inv1 = lax.rsqrt(v1 + EPS)
    sc1 = vecs_ref[0, 2:3, :] * inv1
    sh1 = vecs_ref[0, 3:4, :] - m1 * sc1

    t1 = jnp.zeros((1, H), f32)
    t2 = jnp.zeros((1, H), f32)
    for b in range(B):
        for cj in range(CPB):
            r0 = PAD + cj * C
            y = (jnp.maximum(out_ref[b, cj * C:(cj + 1) * C, :] * sc1 + sh1,
                             0.0)
                 + hb_s[b, r0:r0 + C, :].astype(f32))
            out_ref[b, cj * C:(cj + 1) * C, :] = y
            t1 = t1 + jnp.dot(ones_row, y, preferred_element_type=f32)
            t2 = t2 + jnp.dot(ones_row, y * y, preferred_element_type=f32)

    m2 = t1 * (1.0 / N)
    v2 = t2 * (1.0 / N) - m2 * m2
    inv2 = lax.rsqrt(v2 + EPS)
    sc2 = vecs_ref[0, 4:5, :] * inv2
    sh2 = vecs_ref[0, 5:6, :] - m2 * sc2

    for b in range(B):
        for cj in range(CPB):
            r0 = PAD + cj * C
            z = out_ref[b, cj * C:(cj + 1) * C, :] * sc2 + sh2
            out_ref[b, cj * C:(cj + 1) * C, :] = z
            hb_s[b, r0:r0 + C, :] = z.astype(jnp.bfloat16)


def kernel(n_coords, ca_coords, c_coords, params):
    f32 = jnp.float32
    bf16 = jnp.bfloat16
    ca = ca_coords.astype(f32)
    wproj = jnp.concatenate([params["W_proj"].T.astype(f32),
                             params["b_proj"][None, :].astype(f32),
                             jnp.zeros((4, H), f32)], axis=0)
    wl = jnp.stack([params[f"W_lin{i}"].astype(bf16)
                    for i in range(NUM_LAYERS)])
    ws = jnp.stack([params[f"W_self{i}"].astype(bf16)
                    for i in range(NUM_LAYERS)])
    z = jnp.zeros((H,), f32)
    vecs = jnp.stack([
        jnp.stack([params[f"b_lin{i}"], params[f"b_self{i}"],
                   params[f"g_in{i}"], params[f"b_in{i}"],
                   params[f"g_out{i}"], params[f"b_out{i}"], z, z]).astype(f32)
        for i in range(NUM_LAYERS)])

    return pl.pallas_call(
        _gear_body,
        grid=(NUM_LAYERS,),
        in_specs=[
            pl.BlockSpec((B, L, 3), lambda l: (0, 0, 0)),
            pl.BlockSpec((8, H), lambda l: (0, 0)),
            pl.BlockSpec((1, H, R * H), lambda l: (l, 0, 0)),
            pl.BlockSpec((1, H, H), lambda l: (l, 0, 0)),
            pl.BlockSpec((1, 8, H), lambda l: (l, 0, 0)),
        ],
        out_specs=pl.BlockSpec((B, L, H), lambda l: (0, 0, 0)),
        out_shape=jax.ShapeDtypeStruct((B, L, H), f32),
        scratch_shapes=[
            pltpu.VMEM((B, PL_ROWS, H), jnp.bfloat16),
            pltpu.VMEM((B, L, L), jnp.bfloat16),
            pltpu.VMEM((L, H), jnp.bfloat16),
        ],
        compiler_params=pltpu.CompilerParams(
            dimension_semantics=("arbitrary",),
            vmem_limit_bytes=64 * 1024 * 1024,
        ),
    )(ca, wproj, wl, ws, vecs)


# speedup vs baseline: 1.3343x; 1.0254x over previous
import jax
import jax.numpy as jnp
from jax import lax
from jax.experimental import pallas as pl
from jax.experimental.pallas import tpu as pltpu

B, L, H, R, K = 4, 1024, 512, 7, 10
N = B * L
NUM_LAYERS = 4
PAD = 8
PL_ROWS = L + 2 * PAD
OFFSETS = (-3, -2, -1, 1, 2, 3)
C = 512
CPB = L // C
TR = 128
ACH = L // TR
EPS = 1e-5
BIG = 3.0e38

_DNT = (((1,), (1,)), ((), ()))


def _gear_body(ca_ref, wproj_ref, wl_ref, ws_ref, vecs_ref, out_ref,
               hb_s, at_s, p6_s):
    l = pl.program_id(0)
    f32 = jnp.float32
    bf16 = jnp.bfloat16

    @pl.when(l == 0)
    def _init():
        iot0 = lax.broadcasted_iota(jnp.int32, (L, TR), 0)
        iot1 = lax.broadcasted_iota(jnp.int32, (L, TR), 1)

        def _per_batch(b, carry):
            x3 = ca_ref[b]
            xo = jnp.concatenate(
                [x3, jnp.ones((L, 1), f32), jnp.zeros((L, 4), f32)], axis=1)
            h0 = jnp.dot(xo, wproj_ref[...], preferred_element_type=f32)
            hb_s[b, 0:PAD, :] = jnp.zeros((PAD, H), bf16)
            hb_s[b, PAD + L:PL_ROWS, :] = jnp.zeros((PAD, H), bf16)
            hb_s[b, PAD:PAD + L, :] = h0.astype(bf16)

            rows = [jnp.transpose(x3[:, cd:cd + 1]) for cd in range(3)]

            for ci in range(ACH):
                i0 = ci * TR
                d2 = jnp.zeros((L, TR), f32)
                for cd in range(3):
                    col = x3[:, cd:cd + 1]
                    row = rows[cd][:, i0:i0 + TR]
                    df = col - row
                    d2 = d2 + df * df
                self_oh = iot0 == (iot1 + i0)
                d = jnp.where(self_oh, BIG, jnp.sqrt(d2))
                am = jnp.full((1, TR), -1, jnp.int32)
                for t in range(K):
                    d = jnp.where(iot0 == am, BIG, d)
                    m = jnp.min(d, axis=0, keepdims=True)
                    sel = jnp.where(d == m, iot0, L)
                    am = jnp.min(sel, axis=0, keepdims=True)
                at = ((d == BIG) & jnp.logical_not(self_oh)) | (iot0 == am)
                at_s[b, :, i0:i0 + TR] = at.astype(bf16)
            return carry

        lax.fori_loop(0, B, _per_batch, 0)

    ones_row = jnp.ones((1, C), f32)
    bias = vecs_ref[0, 0:1, :] + vecs_ref[0, 1:2, :]

    s1 = jnp.zeros((1, H), f32)
    s2 = jnp.zeros((1, H), f32)
    for b in range(B):
        p6_s[...] = lax.dot_general(
            hb_s[b, PAD:PAD + L, :], wl_ref[0, :, 6 * H:7 * H], _DNT,
            preferred_element_type=f32).astype(jnp.bfloat16)
        for cj in range(CPB):
            r0 = PAD + cj * C
            acc = lax.dot_general(hb_s[b, r0:r0 + C, :], ws_ref[0], _DNT,
                                  preferred_element_type=f32) + bias
            for r, off in enumerate(OFFSETS):
                acc = acc + lax.dot_general(
                    hb_s[b, r0 - off:r0 - off + C, :],
                    wl_ref[0, :, r * H:(r + 1) * H], _DNT,
                    preferred_element_type=f32)
            acc = acc + jnp.dot(at_s[b, cj * C:(cj + 1) * C, :], p6_s[...],
                                preferred_element_type=f32)
            out_ref[b, cj * C:(cj + 1) * C, :] = acc
            s1 = s1 + jnp.dot(ones_row, acc, preferred_element_type=f32)
            s2 = s2 + jnp.dot(ones_row, acc * acc, preferred_element_type=f32)

    m1 = s1 * (1.0 / N)
    v1 = s2 * (1.0 / N) - m1 * m1
    inv1 = lax.rsqrt(v1 + EPS)
    sc1 = vecs_ref[0, 2:3, :] * inv1
    sh1 = vecs_ref[0, 3:4, :] - m1 * sc1

    t1 = jnp.zeros((1, H), f32)
    t2 = jnp.zeros((1, H), f32)
    for b in range(B):
        for cj in range(CPB):
            r0 = PAD + cj * C
            y = (jnp.maximum(out_ref[b, cj * C:(cj + 1) * C, :] * sc1 + sh1,
                             0.0)
                 + hb_s[b, r0:r0 + C, :].astype(f32))
            out_ref[b, cj * C:(cj + 1) * C, :] = y
            t1 = t1 + jnp.dot(ones_row, y, preferred_element_type=f32)
            t2 = t2 + jnp.dot(ones_row, y * y, preferred_element_type=f32)

    m2 = t1 * (1.0 / N)
    v2 = t2 * (1.0 / N) - m2 * m2
    inv2 = lax.rsqrt(v2 + EPS)
    sc2 = vecs_ref[0, 4:5, :] * inv2
    sh2 = vecs_ref[0, 5:6, :] - m2 * sc2

    for b in range(B):
        for cj in range(CPB):
            r0 = PAD + cj * C
            z = out_ref[b, cj * C:(cj + 1) * C, :] * sc2 + sh2
            out_ref[b, cj * C:(cj + 1) * C, :] = z
            hb_s[b, r0:r0 + C, :] = z.astype(jnp.bfloat16)


def kernel(n_coords, ca_coords, c_coords, params):
    f32 = jnp.float32
    bf16 = jnp.bfloat16
    ca = ca_coords.astype(f32)
    wproj = jnp.concatenate([params["W_proj"].T.astype(f32),
                             params["b_proj"][None, :].astype(f32),
                             jnp.zeros((4, H), f32)], axis=0)
    wl = jnp.stack([params[f"W_lin{i}"].astype(bf16)
                    for i in range(NUM_LAYERS)])
    ws = jnp.stack([params[f"W_self{i}"].astype(bf16)
                    for i in range(NUM_LAYERS)])
    z = jnp.zeros((H,), f32)
    vecs = jnp.stack([
        jnp.stack([params[f"b_lin{i}"], params[f"b_self{i}"],
                   params[f"g_in{i}"], params[f"b_in{i}"],
                   params[f"g_out{i}"], params[f"b_out{i}"], z, z]).astype(f32)
        for i in range(NUM_LAYERS)])

    return pl.pallas_call(
        _gear_body,
        grid=(NUM_LAYERS,),
        in_specs=[
            pl.BlockSpec((B, L, 3), lambda l: (0, 0, 0)),
            pl.BlockSpec((8, H), lambda l: (0, 0)),
            pl.BlockSpec((1, H, R * H), lambda l: (l, 0, 0)),
            pl.BlockSpec((1, H, H), lambda l: (l, 0, 0)),
            pl.BlockSpec((1, 8, H), lambda l: (l, 0, 0)),
        ],
        out_specs=pl.BlockSpec((B, L, H), lambda l: (0, 0, 0)),
        out_shape=jax.ShapeDtypeStruct((B, L, H), f32),
        scratch_shapes=[
            pltpu.VMEM((B, PL_ROWS, H), jnp.bfloat16),
            pltpu.VMEM((B, L, L), jnp.bfloat16),
            pltpu.VMEM((L, H), jnp.bfloat16),
        ],
        compiler_params=pltpu.CompilerParams(
            dimension_semantics=("arbitrary",),
            vmem_limit_bytes=64 * 1024 * 1024,
        ),
    )(ca, wproj, wl, ws, vecs)


# speedup vs baseline: 1.7249x; 1.2928x over previous
import jax
import jax.numpy as jnp
from jax import lax
from jax.experimental import pallas as pl
from jax.experimental.pallas import tpu as pltpu

B, L, H, R, K = 4, 1024, 512, 7, 10
N = B * L
NUM_LAYERS = 4
PAD = 8
PL_ROWS = L + 2 * PAD
OFFSETS = (-3, -2, -1, 1, 2, 3)
C = 512
CPB = L // C
TR = 256
ACH = L // TR
EPS = 1e-5
BIG = 3.0e38

_DNT = (((1,), (1,)), ((), ()))


def _gear_body(ca_ref, wproj_ref, wl_ref, ws_ref, vecs_ref, out_ref,
               hb_s, at_s, p6_s):
    l = pl.program_id(0)
    f32 = jnp.float32
    bf16 = jnp.bfloat16

    @pl.when(l == 0)
    def _init():
        iot0 = lax.broadcasted_iota(jnp.int32, (L, TR), 0)
        iot1 = lax.broadcasted_iota(jnp.int32, (L, TR), 1)

        def _per_batch(b, carry):
            x3 = ca_ref[b]
            xo = jnp.concatenate(
                [x3, jnp.ones((L, 1), f32), jnp.zeros((L, 4), f32)], axis=1)
            h0 = jnp.dot(xo, wproj_ref[...], preferred_element_type=f32)
            hb_s[b, 0:PAD, :] = jnp.zeros((PAD, H), bf16)
            hb_s[b, PAD + L:PL_ROWS, :] = jnp.zeros((PAD, H), bf16)
            hb_s[b, PAD:PAD + L, :] = h0.astype(bf16)

            rows = [jnp.transpose(x3[:, cd:cd + 1]) for cd in range(3)]

            for ci in range(ACH):
                i0 = ci * TR
                d2 = jnp.zeros((L, TR), f32)
                for cd in range(3):
                    col = x3[:, cd:cd + 1]
                    row = rows[cd][:, i0:i0 + TR]
                    df = col - row
                    d2 = d2 + df * df
                self_oh = iot0 == (iot1 + i0)
                d = jnp.where(self_oh, BIG, jnp.sqrt(d2))
                am = jnp.full((1, TR), -1, jnp.int32)
                for t in range(K):
                    d = jnp.where(iot0 == am, BIG, d)
                    m = jnp.min(d, axis=0, keepdims=True)
                    sel = jnp.where(d == m, iot0, L)
                    am = jnp.min(sel, axis=0, keepdims=True)
                at = ((d == BIG) & jnp.logical_not(self_oh)) | (iot0 == am)
                at_s[b, :, i0:i0 + TR] = at.astype(bf16)
            return carry

        lax.fori_loop(0, B, _per_batch, 0)

    ones_row = jnp.ones((1, C), f32)
    bias = vecs_ref[0, 0:1, :] + vecs_ref[0, 1:2, :]

    s1 = jnp.zeros((1, H), f32)
    s2 = jnp.zeros((1, H), f32)
    for b in range(B):
        p6_s[...] = lax.dot_general(
            hb_s[b, PAD:PAD + L, :], wl_ref[0, :, 6 * H:7 * H], _DNT,
            preferred_element_type=f32).astype(jnp.bfloat16)
        for cj in range(CPB):
            r0 = PAD + cj * C
            acc = lax.dot_general(hb_s[b, r0:r0 + C, :], ws_ref[0], _DNT,
                                  preferred_element_type=f32) + bias
            for r, off in enumerate(OFFSETS):
                acc = acc + lax.dot_general(
                    hb_s[b, r0 - off:r0 - off + C, :],
                    wl_ref[0, :, r * H:(r + 1) * H], _DNT,
                    preferred_element_type=f32)
            acc = acc + jnp.dot(at_s[b, cj * C:(cj + 1) * C, :], p6_s[...],
                                preferred_element_type=f32)
            out_ref[b, cj * C:(cj + 1) * C, :] = acc
            s1 = s1 + jnp.dot(ones_row, acc, preferred_element_type=f32)
            s2 = s2 + jnp.dot(ones_row, acc * acc, preferred_element_type=f32)

    m1 = s1 * (1.0 / N)
    v1 = s2 * (1.0 / N) - m1 * m1
    inv1 = lax.rsqrt(v1 + EPS)
    sc1 = vecs_ref[0, 2:3, :] * inv1
    sh1 = vecs_ref[0, 3:4, :] - m1 * sc1

    t1 = jnp.zeros((1, H), f32)
    t2 = jnp.zeros((1, H), f32)
    for b in range(B):
        for cj in range(CPB):
            r0 = PAD + cj * C
            y = (jnp.maximum(out_ref[b, cj * C:(cj + 1) * C, :] * sc1 + sh1,
                             0.0)
                 + hb_s[b, r0:r0 + C, :].astype(f32))
            out_ref[b, cj * C:(cj + 1) * C, :] = y
            t1 = t1 + jnp.dot(ones_row, y, preferred_element_type=f32)
            t2 = t2 + jnp.dot(ones_row, y * y, preferred_element_type=f32)

    m2 = t1 * (1.0 / N)
    v2 = t2 * (1.0 / N) - m2 * m2
    inv2 = lax.rsqrt(v2 + EPS)
    sc2 = vecs_ref[0, 4:5, :] * inv2
    sh2 = vecs_ref[0, 5:6, :] - m2 * sc2

    for b in range(B):
        for cj in range(CPB):
            r0 = PAD + cj * C
            z = out_ref[b, cj * C:(cj + 1) * C, :] * sc2 + sh2
            out_ref[b, cj * C:(cj + 1) * C, :] = z
            hb_s[b, r0:r0 + C, :] = z.astype(jnp.bfloat16)


def kernel(n_coords, ca_coords, c_coords, params):
    f32 = jnp.float32
    bf16 = jnp.bfloat16
    ca = ca_coords.astype(f32)
    wproj = jnp.concatenate([params["W_proj"].T.astype(f32),
                             params["b_proj"][None, :].astype(f32),
                             jnp.zeros((4, H), f32)], axis=0)
    wl = jnp.stack([params[f"W_lin{i}"].astype(bf16)
                    for i in range(NUM_LAYERS)])
    ws = jnp.stack([params[f"W_self{i}"].astype(bf16)
                    for i in range(NUM_LAYERS)])
    z = jnp.zeros((H,), f32)
    vecs = jnp.stack([
        jnp.stack([params[f"b_lin{i}"], params[f"b_self{i}"],
                   params[f"g_in{i}"], params[f"b_in{i}"],
                   params[f"g_out{i}"], params[f"b_out{i}"], z, z]).astype(f32)
        for i in range(NUM_LAYERS)])

    return pl.pallas_call(
        _gear_body,
        grid=(NUM_LAYERS,),
        in_specs=[
            pl.BlockSpec((B, L, 3), lambda l: (0, 0, 0)),
            pl.BlockSpec((8, H), lambda l: (0, 0)),
            pl.BlockSpec((1, H, R * H), lambda l: (l, 0, 0)),
            pl.BlockSpec((1, H, H), lambda l: (l, 0, 0)),
            pl.BlockSpec((1, 8, H), lambda l: (l, 0, 0)),
        ],
        out_specs=pl.BlockSpec((B, L, H), lambda l: (0, 0, 0)),
        out_shape=jax.ShapeDtypeStruct((B, L, H), f32),
        scratch_shapes=[
            pltpu.VMEM((B, PL_ROWS, H), jnp.bfloat16),
            pltpu.VMEM((B, L, L), jnp.bfloat16),
            pltpu.VMEM((L, H), jnp.bfloat16),
        ],
        compiler_params=pltpu.CompilerParams(
            dimension_semantics=("arbitrary",),
            vmem_limit_bytes=64 * 1024 * 1024,
        ),
    )(ca, wproj, wl, ws, vecs)


# speedup vs baseline: 1.7359x; 1.0064x over previous
import jax
import jax.numpy as jnp
from jax import lax
from jax.experimental import pallas as pl
from jax.experimental.pallas import tpu as pltpu

B, L, H, R, K = 4, 1024, 512, 7, 10
N = B * L
NUM_LAYERS = 4
PAD = 8
PL_ROWS = L + 2 * PAD
OFFSETS = (-3, -2, -1, 1, 2, 3)
C = 512
CPB = L // C
TR = 512
ACH = L // TR
EPS = 1e-5
BIG = 3.0e38

_DNT = (((1,), (1,)), ((), ()))


def _gear_body(ca_ref, wproj_ref, wl_ref, ws_ref, vecs_ref, out_ref,
               hb_s, at_s, p6_s):
    l = pl.program_id(0)
    f32 = jnp.float32
    bf16 = jnp.bfloat16

    @pl.when(l == 0)
    def _init():
        iot0 = lax.broadcasted_iota(jnp.int32, (L, TR), 0)
        iot1 = lax.broadcasted_iota(jnp.int32, (L, TR), 1)

        def _per_batch(b, carry):
            x3 = ca_ref[b]
            xo = jnp.concatenate(
                [x3, jnp.ones((L, 1), f32), jnp.zeros((L, 4), f32)], axis=1)
            h0 = jnp.dot(xo, wproj_ref[...], preferred_element_type=f32)
            hb_s[b, 0:PAD, :] = jnp.zeros((PAD, H), bf16)
            hb_s[b, PAD + L:PL_ROWS, :] = jnp.zeros((PAD, H), bf16)
            hb_s[b, PAD:PAD + L, :] = h0.astype(bf16)

            rows = [jnp.transpose(x3[:, cd:cd + 1]) for cd in range(3)]

            for ci in range(ACH):
                i0 = ci * TR
                d2 = jnp.zeros((L, TR), f32)
                for cd in range(3):
                    col = x3[:, cd:cd + 1]
                    row = rows[cd][:, i0:i0 + TR]
                    df = col - row
                    d2 = d2 + df * df
                self_oh = iot0 == (iot1 + i0)
                d = jnp.where(self_oh, BIG, jnp.sqrt(d2))
                am = jnp.full((1, TR), -1, jnp.int32)
                for t in range(K):
                    d = jnp.where(iot0 == am, BIG, d)
                    m = jnp.min(d, axis=0, keepdims=True)
                    sel = jnp.where(d == m, iot0, L)
                    am = jnp.min(sel, axis=0, keepdims=True)
                at = ((d == BIG) & jnp.logical_not(self_oh)) | (iot0 == am)
                at_s[b, :, i0:i0 + TR] = at.astype(bf16)
            return carry

        lax.fori_loop(0, B, _per_batch, 0)

    ones_row = jnp.ones((1, C), f32)
    bias = vecs_ref[0, 0:1, :] + vecs_ref[0, 1:2, :]

    s1 = jnp.zeros((1, H), f32)
    s2 = jnp.zeros((1, H), f32)
    for b in range(B):
        p6_s[...] = lax.dot_general(
            hb_s[b, PAD:PAD + L, :], wl_ref[0, :, 6 * H:7 * H], _DNT,
            preferred_element_type=f32).astype(jnp.bfloat16)
        for cj in range(CPB):
            r0 = PAD + cj * C
            acc = lax.dot_general(hb_s[b, r0:r0 + C, :], ws_ref[0], _DNT,
                                  preferred_element_type=f32) + bias
            for r, off in enumerate(OFFSETS):
                acc = acc + lax.dot_general(
                    hb_s[b, r0 - off:r0 - off + C, :],
                    wl_ref[0, :, r * H:(r + 1) * H], _DNT,
                    preferred_element_type=f32)
            acc = acc + jnp.dot(at_s[b, cj * C:(cj + 1) * C, :], p6_s[...],
                                preferred_element_type=f32)
            out_ref[b, cj * C:(cj + 1) * C, :] = acc
            s1 = s1 + jnp.dot(ones_row, acc, preferred_element_type=f32)
            s2 = s2 + jnp.dot(ones_row, acc * acc, preferred_element_type=f32)

    m1 = s1 * (1.0 / N)
    v1 = s2 * (1.0 / N) - m1 * m1
    inv1 = lax.rsqrt(v1 + EPS)
    sc1 = vecs_ref[0, 2:3, :] * inv1
    sh1 = vecs_ref[0, 3:4, :] - m1 * sc1

    t1 = jnp.zeros((1, H), f32)
    t2 = jnp.zeros((1, H), f32)
    for b in range(B):
        for cj in range(CPB):
            r0 = PAD + cj * C
            y = (jnp.maximum(out_ref[b, cj * C:(cj + 1) * C, :] * sc1 + sh1,
                             0.0)
                 + hb_s[b, r0:r0 + C, :].astype(f32))
            out_ref[b, cj * C:(cj + 1) * C, :] = y
            t1 = t1 + jnp.dot(ones_row, y, preferred_element_type=f32)
            t2 = t2 + jnp.dot(ones_row, y * y, preferred_element_type=f32)

    m2 = t1 * (1.0 / N)
    v2 = t2 * (1.0 / N) - m2 * m2
    inv2 = lax.rsqrt(v2 + EPS)
    sc2 = vecs_ref[0, 4:5, :] * inv2
    sh2 = vecs_ref[0, 5:6, :] - m2 * sc2

    for b in range(B):
        for cj in range(CPB):
            r0 = PAD + cj * C
            z = out_ref[b, cj * C:(cj + 1) * C, :] * sc2 + sh2
            out_ref[b, cj * C:(cj + 1) * C, :] = z
            hb_s[b, r0:r0 + C, :] = z.astype(jnp.bfloat16)


def kernel(n_coords, ca_coords, c_coords, params):
    f32 = jnp.float32
    bf16 = jnp.bfloat16
    ca = ca_coords.astype(f32)
    wproj = jnp.concatenate([params["W_proj"].T.astype(f32),
                             params["b_proj"][None, :].astype(f32),
                             jnp.zeros((4, H), f32)], axis=0)
    wl = jnp.stack([params[f"W_lin{i}"].astype(bf16)
                    for i in range(NUM_LAYERS)])
    ws = jnp.stack([params[f"W_self{i}"].astype(bf16)
                    for i in range(NUM_LAYERS)])
    z = jnp.zeros((H,), f32)
    vecs = jnp.stack([
        jnp.stack([params[f"b_lin{i}"], params[f"b_self{i}"],
                   params[f"g_in{i}"], params[f"b_in{i}"],
                   params[f"g_out{i}"], params[f"b_out{i}"], z, z]).astype(f32)
        for i in range(NUM_LAYERS)])

    return pl.pallas_call(
        _gear_body,
        grid=(NUM_LAYERS,),
        in_specs=[
            pl.BlockSpec((B, L, 3), lambda l: (0, 0, 0)),
            pl.BlockSpec((8, H), lambda l: (0, 0)),
            pl.BlockSpec((1, H, R * H), lambda l: (l, 0, 0)),
            pl.BlockSpec((1, H, H), lambda l: (l, 0, 0)),
            pl.BlockSpec((1, 8, H), lambda l: (l, 0, 0)),
        ],
        out_specs=pl.BlockSpec((B, L, H), lambda l: (0, 0, 0)),
        out_shape=jax.ShapeDtypeStruct((B, L, H), f32),
        scratch_shapes=[
            pltpu.VMEM((B, PL_ROWS, H), jnp.bfloat16),
            pltpu.VMEM((B, L, L), jnp.bfloat16),
            pltpu.VMEM((L, H), jnp.bfloat16),
        ],
        compiler_params=pltpu.CompilerParams(
            dimension_semantics=("arbitrary",),
            vmem_limit_bytes=64 * 1024 * 1024,
        ),
    )(ca, wproj, wl, ws, vecs)
